# baseline (speedup 1.0000x reference)
import jax
import jax.numpy as jnp
from jax.experimental import pallas as pl
from jax.experimental.pallas import tpu as pltpu

NUM_GENES = 64
NUM_PEAK = 128
NUM_TF = 64
GENE_DIM = 2
E = 1024

BT = 8
TFC = 8


def _subnet_kernel(x_ref, w_ref, b_ref, redm_ref, out_ref, acc_ref):
    j = pl.program_id(1)
    half = NUM_TF // 2
    part = x_ref[:, 0:TFC, :] * w_ref[0:TFC, :][None]
    for c in range(1, half // TFC):
        part = part + x_ref[:, c * TFC:(c + 1) * TFC, :] * w_ref[c * TFC:(c + 1) * TFC, :][None]

    @pl.when(j == 0)
    def _():
        acc_ref[...] = part

    @pl.when(j > 0)
    def _():
        acc_ref[...] = acc_ref[...] + part

    @pl.when(j == pl.num_programs(1) - 1)
    def _():
        t = jnp.sum(acc_ref[...], axis=1)
        s = jnp.dot(t, redm_ref[...], preferred_element_type=jnp.float32)
        out_ref[...] = jnp.maximum(s + b_ref[...], 0.0)


def _graph_kernel(xc_ref, ei_ref, eit_ref, cw_ref, cb_ref, owt_ref, ob_ref,
                  hf_ref, out_ref):
    xc = xc_ref[...]
    src_r = ei_ref[0:1, :]
    dst_r = ei_ref[1:2, :]
    dst_c = eit_ref[:, 1:2]

    gid_r = jax.lax.broadcasted_iota(jnp.int32, (NUM_GENES, E), 0)
    gid_c = jax.lax.broadcasted_iota(jnp.int32, (E, NUM_GENES), 1)

    mdst = (dst_r == gid_r).astype(jnp.float32)
    mdst_t = (dst_c == gid_c).astype(jnp.float32)
    msrc = (src_r == gid_r).astype(jnp.float32)

    deg_c = jnp.sum(mdst, axis=1, keepdims=True)
    deg_r = jnp.sum(mdst_t, axis=0, keepdims=True)
    dinv_c = jnp.where(deg_c > 0, jax.lax.rsqrt(jnp.maximum(deg_c, 1.0)), 0.0)
    dinv_r = jnp.where(deg_r > 0, jax.lax.rsqrt(jnp.maximum(deg_r, 1.0)), 0.0)

    ms = msrc * dinv_c
    mdt = mdst_t * dinv_r

    proj = jnp.dot(xc, ms, preferred_element_type=jnp.float32)
    t = jnp.dot(proj, mdt, preferred_element_type=jnp.float32)

    cw0 = cw_ref[0, 0]
    cw1 = cw_ref[0, 1]
    cb0 = cb_ref[0, 0]
    cb1 = cb_ref[0, 1]
    l_i = jax.lax.broadcasted_iota(jnp.int32, (NUM_GENES, GENE_DIM * NUM_GENES), 1)
    r_i = jax.lax.broadcasted_iota(jnp.int32, (NUM_GENES, GENE_DIM * NUM_GENES), 0)
    rep = jnp.where(l_i // GENE_DIM == r_i,
                    jnp.where(l_i % GENE_DIM == 0, cw0, cw1), 0.0)
    lb = jax.lax.broadcasted_iota(jnp.int32, (1, GENE_DIM * NUM_GENES), 1)
    cbvec = jnp.where(lb % GENE_DIM == 0, cb0, cb1)

    hf = jnp.maximum(jnp.dot(t, rep, preferred_element_type=jnp.float32) + cbvec,
                     0.0)
    hf_ref[...] = hf
    out_ref[...] = (jnp.dot(hf, owt_ref[...], preferred_element_type=jnp.float32)
                    + ob_ref[...])


def _run(x3, wt, b2, redm, ei, eit, cw, cb, owt, ob):
    Bn = x3.shape[0]
    L = NUM_GENES * NUM_PEAK
    x_cat = pl.pallas_call(
        _subnet_kernel,
        grid=(Bn // BT, 2),
        in_specs=[
            pl.BlockSpec((BT, NUM_TF // 2, L), lambda i, j: (i, j, 0)),
            pl.BlockSpec((NUM_TF // 2, L), lambda i, j: (j, 0)),
            pl.BlockSpec((1, NUM_GENES), lambda i, j: (0, 0)),
            pl.BlockSpec((L, NUM_GENES), lambda i, j: (0, 0)),
        ],
        out_specs=pl.BlockSpec((BT, NUM_GENES), lambda i, j: (i, 0)),
        out_shape=jax.ShapeDtypeStruct((Bn, NUM_GENES), jnp.float32),
        scratch_shapes=[pltpu.VMEM((BT, TFC, L), jnp.float32)],
    )(x3, wt, b2, redm)

    hf, out = pl.pallas_call(
        _graph_kernel,
        in_specs=[
            pl.BlockSpec(memory_space=pltpu.VMEM),
            pl.BlockSpec(memory_space=pltpu.VMEM),
            pl.BlockSpec(memory_space=pltpu.VMEM),
            pl.BlockSpec(memory_space=pltpu.SMEM),
            pl.BlockSpec(memory_space=pltpu.SMEM),
            pl.BlockSpec(memory_space=pltpu.VMEM),
            pl.BlockSpec(memory_space=pltpu.VMEM),
        ],
        out_specs=[
            pl.BlockSpec(memory_space=pltpu.VMEM),
            pl.BlockSpec(memory_space=pltpu.VMEM),
        ],
        out_shape=[
            jax.ShapeDtypeStruct((Bn, GENE_DIM * NUM_GENES), jnp.float32),
            jax.ShapeDtypeStruct((Bn, 3), jnp.float32),
        ],
    )(x_cat, ei, eit, cw, cb, owt, ob)
    return x_cat, hf, out


def kernel(x, sub_W, sub_b, conv_W, conv_b, out_W, out_b, edge_index):
    Bn = x.shape[0]
    L = NUM_GENES * NUM_PEAK
    wt = jnp.transpose(sub_W.reshape(NUM_GENES, NUM_TF, NUM_PEAK),
                       (1, 0, 2)).reshape(NUM_TF, L)
    b2 = sub_b.reshape(1, NUM_GENES)
    lane_g = jnp.arange(L, dtype=jnp.int32) // NUM_PEAK
    redm = (lane_g[:, None] == jnp.arange(NUM_GENES, dtype=jnp.int32)[None, :]
            ).astype(jnp.float32)
    ei = edge_index.astype(jnp.int32)
    eit = ei.T
    cw = conv_W.reshape(1, GENE_DIM)
    cb = conv_b.reshape(1, GENE_DIM)
    owt = out_W.T
    ob = out_b.reshape(1, 3)
    return _run(x, wt, b2, redm, ei, eit, cw, cb, owt, ob)


# speedup vs baseline: 1.0754x; 1.0754x over previous
import jax
import jax.numpy as jnp
from jax.experimental import pallas as pl
from jax.experimental.pallas import tpu as pltpu

NUM_GENES = 64
NUM_PEAK = 128
NUM_TF = 64
GENE_DIM = 2
E = 1024

BT = 8
TFC = 8


def _subnet_kernel(xa_ref, xb_ref, w_ref, b_ref, redm_ref, out_ref):
    half = NUM_TF // 2
    acc = xa_ref[:, 0:TFC, :] * w_ref[0:TFC, :][None]
    for c in range(1, half // TFC):
        acc = acc + xa_ref[:, c * TFC:(c + 1) * TFC, :] * w_ref[c * TFC:(c + 1) * TFC, :][None]
    for c in range(half // TFC):
        acc = acc + xb_ref[:, c * TFC:(c + 1) * TFC, :] * w_ref[half + c * TFC:half + (c + 1) * TFC, :][None]
    t = jnp.sum(acc, axis=1)
    s = jnp.dot(t, redm_ref[...], preferred_element_type=jnp.float32)
    out_ref[...] = jnp.maximum(s + b_ref[...], 0.0)


def _graph_kernel(xc_ref, ei_ref, eit_ref, cw_ref, cb_ref, owt_ref, ob_ref,
                  hf_ref, out_ref):
    xc = xc_ref[...]
    src_r = ei_ref[0:1, :]
    dst_r = ei_ref[1:2, :]
    dst_c = eit_ref[:, 1:2]

    gid_r = jax.lax.broadcasted_iota(jnp.int32, (NUM_GENES, E), 0)
    gid_c = jax.lax.broadcasted_iota(jnp.int32, (E, NUM_GENES), 1)

    mdst = (dst_r == gid_r).astype(jnp.float32)
    mdst_t = (dst_c == gid_c).astype(jnp.float32)
    msrc = (src_r == gid_r).astype(jnp.float32)

    deg_c = jnp.sum(mdst, axis=1, keepdims=True)
    deg_r = jnp.sum(mdst_t, axis=0, keepdims=True)
    dinv_c = jnp.where(deg_c > 0, jax.lax.rsqrt(jnp.maximum(deg_c, 1.0)), 0.0)
    dinv_r = jnp.where(deg_r > 0, jax.lax.rsqrt(jnp.maximum(deg_r, 1.0)), 0.0)

    ms = msrc * dinv_c
    mdt = mdst_t * dinv_r

    proj = jnp.dot(xc, ms, preferred_element_type=jnp.float32)
    t = jnp.dot(proj, mdt, preferred_element_type=jnp.float32)

    cw0 = cw_ref[0, 0]
    cw1 = cw_ref[0, 1]
    cb0 = cb_ref[0, 0]
    cb1 = cb_ref[0, 1]
    l_i = jax.lax.broadcasted_iota(jnp.int32, (NUM_GENES, GENE_DIM * NUM_GENES), 1)
    r_i = jax.lax.broadcasted_iota(jnp.int32, (NUM_GENES, GENE_DIM * NUM_GENES), 0)
    rep = jnp.where(l_i // GENE_DIM == r_i,
                    jnp.where(l_i % GENE_DIM == 0, cw0, cw1), 0.0)
    lb = jax.lax.broadcasted_iota(jnp.int32, (1, GENE_DIM * NUM_GENES), 1)
    cbvec = jnp.where(lb % GENE_DIM == 0, cb0, cb1)

    hf = jnp.maximum(jnp.dot(t, rep, preferred_element_type=jnp.float32) + cbvec,
                     0.0)
    hf_ref[...] = hf
    out_ref[...] = (jnp.dot(hf, owt_ref[...], preferred_element_type=jnp.float32)
                    + ob_ref[...])


def _run(x3, wt, b2, redm, ei, eit, cw, cb, owt, ob):
    Bn = x3.shape[0]
    L = NUM_GENES * NUM_PEAK
    x_cat = pl.pallas_call(
        _subnet_kernel,
        grid=(Bn // BT,),
        in_specs=[
            pl.BlockSpec((BT, NUM_TF // 2, L), lambda i: (i, 0, 0)),
            pl.BlockSpec((BT, NUM_TF // 2, L), lambda i: (i, 1, 0)),
            pl.BlockSpec((NUM_TF, L), lambda i: (0, 0)),
            pl.BlockSpec((1, NUM_GENES), lambda i: (0, 0)),
            pl.BlockSpec((L, NUM_GENES), lambda i: (0, 0)),
        ],
        out_specs=pl.BlockSpec((BT, NUM_GENES), lambda i: (i, 0)),
        out_shape=jax.ShapeDtypeStruct((Bn, NUM_GENES), jnp.float32),
    )(x3, x3, wt, b2, redm)

    hf, out = pl.pallas_call(
        _graph_kernel,
        in_specs=[
            pl.BlockSpec(memory_space=pltpu.VMEM),
            pl.BlockSpec(memory_space=pltpu.VMEM),
            pl.BlockSpec(memory_space=pltpu.VMEM),
            pl.BlockSpec(memory_space=pltpu.SMEM),
            pl.BlockSpec(memory_space=pltpu.SMEM),
            pl.BlockSpec(memory_space=pltpu.VMEM),
            pl.BlockSpec(memory_space=pltpu.VMEM),
        ],
        out_specs=[
            pl.BlockSpec(memory_space=pltpu.VMEM),
            pl.BlockSpec(memory_space=pltpu.VMEM),
        ],
        out_shape=[
            jax.ShapeDtypeStruct((Bn, GENE_DIM * NUM_GENES), jnp.float32),
            jax.ShapeDtypeStruct((Bn, 3), jnp.float32),
        ],
    )(x_cat, ei, eit, cw, cb, owt, ob)
    return x_cat, hf, out


def kernel(x, sub_W, sub_b, conv_W, conv_b, out_W, out_b, edge_index):
    Bn = x.shape[0]
    L = NUM_GENES * NUM_PEAK
    wt = jnp.transpose(sub_W.reshape(NUM_GENES, NUM_TF, NUM_PEAK),
                       (1, 0, 2)).reshape(NUM_TF, L)
    b2 = sub_b.reshape(1, NUM_GENES)
    lane_g = jnp.arange(L, dtype=jnp.int32) // NUM_PEAK
    redm = (lane_g[:, None] == jnp.arange(NUM_GENES, dtype=jnp.int32)[None, :]
            ).astype(jnp.float32)
    ei = edge_index.astype(jnp.int32)
    eit = ei.T
    cw = conv_W.reshape(1, GENE_DIM)
    cb = conv_b.reshape(1, GENE_DIM)
    owt = out_W.T
    ob = out_b.reshape(1, 3)
    return _run(x, wt, b2, redm, ei, eit, cw, cb, owt, ob)
